# Initial kernel scaffold; baseline (speedup 1.0000x reference)
import jax
import jax.numpy as jnp
from jax.experimental import pallas as pl


def kernel(pocket_x, pocket_pos, ligand_pos, W_embed, b_embed, W_out, b_out):
    raise NotImplementedError("write your pallas kernel here")


# speedup vs baseline: 2.8954x; 2.8954x over previous
import jax
import jax.numpy as jnp
from jax.experimental import pallas as pl
from jax.experimental.pallas import tpu as pltpu

_N = 100000
_NPAD = 100096
_K = 1000
_HID = 128
_OUT = 256
_CH = 4352
_NCH = 23


def _pocket_kernel(lig_ref, xt_ref, post_ref, wet_ref, be_ref, wo_ref,
                   bo_ref, out_ref, mask_ref):
    cx = jnp.sum(lig_ref[0:1, :]) * (1.0 / 32.0)
    cy = jnp.sum(lig_ref[1:2, :]) * (1.0 / 32.0)
    cz = jnp.sum(lig_ref[2:3, :]) * (1.0 / 32.0)
    dx = post_ref[0:1, :] - cx
    dy = post_ref[1:2, :] - cy
    dz = post_ref[2:3, :] - cz
    dist = jnp.sqrt(dx * dx + dy * dy + dz * dz)
    chem = (xt_ref[2:3, :] * 0.3 + xt_ref[3:4, :] * 0.4
            + xt_ref[5:6, :] * 0.3)
    score = jnp.exp(dist * (-1.0 / 8.0)) * 0.7 + chem * 0.3
    idx = jax.lax.broadcasted_iota(jnp.int32, (1, _NPAD), 1)
    score = jnp.where(idx < _N, score, -1.0)
    s = jax.lax.bitcast_convert_type(score, jnp.int32)

    def tbody(i, t):
        cand = t | jnp.left_shift(jnp.int32(1), 30 - i)
        cnt = jnp.sum((s >= cand).astype(jnp.int32))
        return jnp.where(cnt >= _K, cand, t)

    thr = jax.lax.fori_loop(0, 31, tbody, jnp.int32(0))

    m = jnp.sum((s > thr).astype(jnp.int32))
    need = _K - m
    eq = s == thr

    def ibody(i, c):
        cand = c | jnp.left_shift(jnp.int32(1), 16 - i)
        cnt = jnp.sum((eq & (idx < cand)).astype(jnp.int32))
        return jnp.where(cnt < need, cand, c)

    icut = jax.lax.fori_loop(0, 17, ibody, jnp.int32(0))

    sel = (s > thr) | (eq & (idx <= icut))
    mask_ref[...] = sel.astype(jnp.float32)

    wet = wet_ref[...]
    be = be_ref[...]

    def ebody(i, acc):
        off = pl.multiple_of(i * _CH, 128)
        xs = xt_ref[:, pl.ds(off, _CH)]
        mk = mask_ref[:, pl.ds(off, _CH)]
        z = jnp.dot(wet, xs, preferred_element_type=jnp.float32) + be
        z = z * mk
        h = z / (1.0 + jnp.exp(-z))
        return acc + jnp.sum(h, axis=1, keepdims=True)

    acc = jax.lax.fori_loop(0, _NCH, ebody,
                            jnp.zeros((_HID, 1), jnp.float32))
    pooled = acc * (1.0 / _K)
    out = jnp.sum(pooled * wo_ref[...], axis=0, keepdims=True) + bo_ref[...]
    out_ref[...] = out


@jax.jit
def _run(lig, xt, post, wet, be, wo, bo):
    return pl.pallas_call(
        _pocket_kernel,
        out_shape=jax.ShapeDtypeStruct((1, _OUT), jnp.float32),
        scratch_shapes=[pltpu.VMEM((1, _NPAD), jnp.float32)],
    )(lig, xt, post, wet, be, wo, bo)


def kernel(pocket_x, pocket_pos, ligand_pos, W_embed, b_embed, W_out, b_out):
    pad = _NPAD - _N
    xt = jnp.pad(pocket_x.T, ((0, 0), (0, pad)))
    post = jnp.pad(pocket_pos.T, ((0, 0), (0, pad)))
    lig = ligand_pos.T
    wet = W_embed.T
    be = b_embed.reshape(_HID, 1)
    bo = b_out.reshape(1, _OUT)
    out = _run(lig, xt, post, wet, be, wo=W_out, bo=bo)
    return out.reshape(_OUT)


# speedup vs baseline: 3.4517x; 1.1921x over previous
import jax
import jax.numpy as jnp
from jax.experimental import pallas as pl
from jax.experimental.pallas import tpu as pltpu

_N = 100000
_L = 12544
_NPAD = 8 * _L
_K = 1000
_HID = 128
_OUT = 256
_CH = 6272
_NCH = 16


def _pocket_kernel(lig_ref, xc_ref, pr_ref, xt_ref, wet_ref, be_ref, wo_ref,
                   bo_ref, out_ref, mask_ref):
    cx = jnp.sum(lig_ref[0:1, :]) * (1.0 / 32.0)
    cy = jnp.sum(lig_ref[1:2, :]) * (1.0 / 32.0)
    cz = jnp.sum(lig_ref[2:3, :]) * (1.0 / 32.0)
    dx = pr_ref[0:8, :] - cx
    dy = pr_ref[8:16, :] - cy
    dz = pr_ref[16:24, :] - cz
    dist = jnp.sqrt(dx * dx + dy * dy + dz * dz)
    chem = (xc_ref[0:8, :] * 0.3 + xc_ref[8:16, :] * 0.4
            + xc_ref[16:24, :] * 0.3)
    score = jnp.exp(dist * (-1.0 / 8.0)) * 0.7 + chem * 0.3
    idx = (jax.lax.broadcasted_iota(jnp.int32, (8, _L), 0) * _L
           + jax.lax.broadcasted_iota(jnp.int32, (8, _L), 1))
    score = jnp.where(idx < _N, score, -1.0)
    s = jax.lax.bitcast_convert_type(score, jnp.int32)

    def tbody(i, t):
        cand = t | jnp.left_shift(jnp.int32(1), 29 - i)
        cnt = jnp.sum((s >= cand).astype(jnp.int32))
        return jnp.where(cnt >= _K, cand, t)

    thr = jax.lax.fori_loop(0, 30, tbody, jnp.int32(0))

    cnt_ge = jnp.sum((s >= thr).astype(jnp.int32))
    eq = s == thr

    def tie_cut():
        need = _K - (cnt_ge - jnp.sum(eq.astype(jnp.int32)))

        def ibody(i, c):
            cand = c | jnp.left_shift(jnp.int32(1), 16 - i)
            cnt = jnp.sum((eq & (idx < cand)).astype(jnp.int32))
            return jnp.where(cnt < need, cand, c)

        return jax.lax.fori_loop(0, 17, ibody, jnp.int32(0))

    icut = jax.lax.cond(cnt_ge == _K, lambda: jnp.int32(_NPAD), tie_cut)

    sel = (s > thr) | (eq & (idx <= icut))
    mask2 = sel.astype(jnp.float32)
    for r in range(8):
        mask_ref[0:1, r * _L:(r + 1) * _L] = mask2[r:r + 1, :]

    wet = wet_ref[...]
    be = be_ref[...]

    def ebody(i, acc):
        off = pl.multiple_of(i * _CH, 128)
        xs = xt_ref[:, pl.ds(off, _CH)]
        mk = mask_ref[:, pl.ds(off, _CH)]
        z = jnp.dot(wet, xs, preferred_element_type=jnp.float32) + be
        z = z * mk
        h = z / (1.0 + jnp.exp(-z))
        return acc + jnp.sum(h, axis=1, keepdims=True)

    acc = jax.lax.fori_loop(0, _NCH, ebody,
                            jnp.zeros((_HID, 1), jnp.float32))
    pooled = acc * (1.0 / _K)
    out = jnp.sum(pooled * wo_ref[...], axis=0, keepdims=True) + bo_ref[...]
    out_ref[...] = out


@jax.jit
def _run(lig, xc, pr, xt, wet, be, wo, bo):
    return pl.pallas_call(
        _pocket_kernel,
        out_shape=jax.ShapeDtypeStruct((1, _OUT), jnp.float32),
        scratch_shapes=[pltpu.VMEM((1, _NPAD), jnp.float32)],
    )(lig, xc, pr, xt, wet, be, wo, bo)


def kernel(pocket_x, pocket_pos, ligand_pos, W_embed, b_embed, W_out, b_out):
    pad = _NPAD - _N
    xt = jnp.pad(pocket_x.T, ((0, 0), (0, pad)))
    pr = jnp.concatenate(
        [jnp.pad(pocket_pos[:, j], (0, pad)).reshape(8, _L)
         for j in range(3)], axis=0)
    xc = jnp.concatenate(
        [jnp.pad(pocket_x[:, j], (0, pad)).reshape(8, _L)
         for j in (2, 3, 5)], axis=0)
    lig = ligand_pos.T
    wet = W_embed.T
    be = b_embed.reshape(_HID, 1)
    bo = b_out.reshape(1, _OUT)
    out = _run(lig, xc, pr, xt, wet, be, W_out, bo)
    return out.reshape(_OUT)
